# baseline (speedup 1.0000x reference)
import jax
import jax.numpy as jnp
from jax import lax
from jax.experimental import pallas as pl
from jax.experimental.pallas import tpu as pltpu
from jax.experimental.pallas import tpu_sc as plsc

T = 128
K = 8
E = 64
D = 1024
F = 1024
C = F // 2
E_A = 8
LANES = 16
NWORKERS = 32


def _coef_sc_body(ids_hbm, esc_hbm, out_hbm, ids_v, esc_v, acc_v):
    wid = lax.axis_index("s") * 2 + lax.axis_index("c")
    pltpu.sync_copy(ids_hbm, ids_v)
    pltpu.sync_copy(esc_hbm, esc_v)
    zeros = jnp.zeros((LANES,), jnp.float32)
    for i in range(2 * T // LANES):
        acc_v[pl.ds(i * LANES, LANES)] = zeros
    lane = jnp.arange(LANES, dtype=jnp.int32)
    e_base = wid * 2
    for c in range(T * K // LANES):
        iv = ids_v[pl.ds(c * LANES, LANES)]
        sv = esc_v[pl.ds(c * LANES, LANES)]
        t_vec = (c * LANES + lane) >> 3
        for j in range(2):
            mask = iv == (e_base + j)
            plsc.addupdate_scatter(acc_v, [t_vec + j * T], sv, mask=mask)
    pltpu.sync_copy(acc_v, out_hbm.at[pl.ds(e_base * T, 2 * T)])


_coef_sc = pl.kernel(
    _coef_sc_body,
    out_type=jax.ShapeDtypeStruct((E * T,), jnp.float32),
    mesh=plsc.VectorSubcoreMesh(core_axis_name="c", subcore_axis_name="s"),
    compiler_params=pltpu.CompilerParams(needs_layout_passes=False),
    scratch_types=[
        pltpu.VMEM((T * K,), jnp.int32),
        pltpu.VMEM((T * K,), jnp.float32),
        pltpu.VMEM((2 * T,), jnp.float32),
    ],
)


def _expert_ffn(x, e, w1g0_ref, w1g1_ref, w1u0_ref, w1u1_ref, s1_ref,
                w2a_ref, w2b_ref, s2_ref, smooth_ref):
    s1 = s1_ref[0]
    smooth = smooth_ref[0]
    g0 = jnp.dot(x, w1g0_ref[0], preferred_element_type=jnp.float32)
    g1 = jnp.dot(x, w1g1_ref[0], preferred_element_type=jnp.float32)
    u0 = jnp.dot(x, w1u0_ref[0], preferred_element_type=jnp.float32)
    u1 = jnp.dot(x, w1u1_ref[0], preferred_element_type=jnp.float32)
    gate0 = g0 * s1[:, 0:C]
    gate1 = g1 * s1[:, C:F]
    up0 = u0 * s1[:, F:F + C]
    up1 = u1 * s1[:, F + C:]
    act0 = (gate0 * jax.nn.sigmoid(gate0)) * up0 * smooth[:, 0:C]
    act1 = (gate1 * jax.nn.sigmoid(gate1)) * up1 * smooth[:, C:]
    y = jnp.dot(act0, w2a_ref[0], preferred_element_type=jnp.float32)
    y = y + jnp.dot(act1, w2b_ref[0], preferred_element_type=jnp.float32)
    return y * s2_ref[0]


def _moe_a_body(x_ref, ids_ref, escale_ref, w1g0_ref, w1g1_ref, w1u0_ref,
                w1u1_ref, s1_ref, w2a_ref, w2b_ref, s2_ref, smooth_ref,
                out_ref):
    e = pl.program_id(0)
    y = _expert_ffn(x_ref[...], e, w1g0_ref, w1g1_ref, w1u0_ref, w1u1_ref,
                    s1_ref, w2a_ref, w2b_ref, s2_ref, smooth_ref)
    coef = jnp.sum(
        jnp.where(ids_ref[...] == e, escale_ref[...], 0.0), axis=1)
    contrib = coef[:, None] * y

    @pl.when(e == 0)
    def _init():
        out_ref[...] = contrib

    @pl.when(e != 0)
    def _acc():
        out_ref[...] += contrib


def _moe_b_body(x_ref, coef_ref, part_ref, w1g0_ref, w1g1_ref, w1u0_ref,
                w1u1_ref, s1_ref, w2a_ref, w2b_ref, s2_ref, smooth_ref,
                out_ref):
    i = pl.program_id(0)
    y = _expert_ffn(x_ref[...], i, w1g0_ref, w1g1_ref, w1u0_ref, w1u1_ref,
                    s1_ref, w2a_ref, w2b_ref, s2_ref, smooth_ref)
    contrib = coef_ref[0] * y

    @pl.when(i == 0)
    def _init():
        out_ref[...] = part_ref[...] + contrib

    @pl.when(i != 0)
    def _acc():
        out_ref[...] += contrib


def _weight_specs(shift):
    return [
        pl.BlockSpec((1, D, C), lambda e: (e + shift, 0, 0)),
        pl.BlockSpec((1, D, C), lambda e: (e + shift, 0, 1)),
        pl.BlockSpec((1, D, C), lambda e: (e + shift, 0, 2)),
        pl.BlockSpec((1, D, C), lambda e: (e + shift, 0, 3)),
        pl.BlockSpec((1, 1, 2 * F), lambda e: (e + shift, 0, 0)),
        pl.BlockSpec((1, C, D), lambda e: (e + shift, 0, 0)),
        pl.BlockSpec((1, C, D), lambda e: (e + shift, 1, 0)),
        pl.BlockSpec((1, 1, D), lambda e: (e + shift, 0, 0)),
        pl.BlockSpec((1, 1, F), lambda e: (e + shift, 0, 0)),
    ]


def kernel(x, expert_ids, gmm1_weight, gmm1_weight_scale, gmm2_weight,
           gmm2_weight_scale, smooth_scales, expert_scales):
    coef = _coef_sc(expert_ids.reshape(T * K), expert_scales.reshape(T * K))
    coef = coef.reshape(E, T, 1)
    weights = (gmm1_weight, gmm1_weight, gmm1_weight, gmm1_weight,
               gmm1_weight_scale[:, None, :], gmm2_weight, gmm2_weight,
               gmm2_weight_scale[:, None, :], smooth_scales[:, None, :])

    partial = pl.pallas_call(
        _moe_a_body,
        grid=(E_A,),
        in_specs=[
            pl.BlockSpec((T, D), lambda e: (0, 0)),
            pl.BlockSpec((T, K), lambda e: (0, 0)),
            pl.BlockSpec((T, K), lambda e: (0, 0)),
        ] + _weight_specs(0),
        out_specs=pl.BlockSpec((T, D), lambda e: (0, 0)),
        out_shape=jax.ShapeDtypeStruct((T, D), jnp.float32),
    )(x, expert_ids, expert_scales, *weights)

    return pl.pallas_call(
        _moe_b_body,
        grid=(E - E_A,),
        in_specs=[
            pl.BlockSpec((T, D), lambda e: (0, 0)),
            pl.BlockSpec((1, T, 1), lambda e: (e + E_A, 0, 0)),
            pl.BlockSpec((T, D), lambda e: (0, 0)),
        ] + _weight_specs(E_A),
        out_specs=pl.BlockSpec((T, D), lambda e: (0, 0)),
        out_shape=jax.ShapeDtypeStruct((T, D), jnp.float32),
    )(x, coef, partial, *weights)


# speedup vs baseline: 1.1004x; 1.1004x over previous
import jax
import jax.numpy as jnp
from jax.experimental import pallas as pl

T = 128
K = 8
E = 64
D = 1024
F = 1024
C = F // 2


def _moe_body(x_ref, ids_ref, w1g0_ref, w1g1_ref, w1u0_ref, w1u1_ref,
              s1_ref, w2a_ref, w2b_ref, s2_ref, smooth_ref, escale_ref,
              out_ref):
    e = pl.program_id(0)
    x = x_ref[...]
    s1 = s1_ref[0]
    smooth = smooth_ref[0]

    g0 = jnp.dot(x, w1g0_ref[0], preferred_element_type=jnp.float32)
    g1 = jnp.dot(x, w1g1_ref[0], preferred_element_type=jnp.float32)
    u0 = jnp.dot(x, w1u0_ref[0], preferred_element_type=jnp.float32)
    u1 = jnp.dot(x, w1u1_ref[0], preferred_element_type=jnp.float32)

    gate0 = g0 * s1[:, 0:C]
    gate1 = g1 * s1[:, C:F]
    up0 = u0 * s1[:, F:F + C]
    up1 = u1 * s1[:, F + C:]
    act0 = (gate0 * jax.nn.sigmoid(gate0)) * up0 * smooth[:, 0:C]
    act1 = (gate1 * jax.nn.sigmoid(gate1)) * up1 * smooth[:, C:]

    y = jnp.dot(act0, w2a_ref[0], preferred_element_type=jnp.float32)
    y = y + jnp.dot(act1, w2b_ref[0], preferred_element_type=jnp.float32)
    y = y * s2_ref[0]

    coef = jnp.sum(
        jnp.where(ids_ref[...] == e, escale_ref[...], 0.0), axis=1)
    contrib = coef[:, None] * y

    @pl.when(e == 0)
    def _init():
        out_ref[...] = contrib

    @pl.when(e != 0)
    def _acc():
        out_ref[...] += contrib


def kernel(x, expert_ids, gmm1_weight, gmm1_weight_scale, gmm2_weight,
           gmm2_weight_scale, smooth_scales, expert_scales):
    return pl.pallas_call(
        _moe_body,
        grid=(E,),
        in_specs=[
            pl.BlockSpec((T, D), lambda e: (0, 0)),
            pl.BlockSpec((T, K), lambda e: (0, 0)),
            pl.BlockSpec((1, D, C), lambda e: (e, 0, 0)),
            pl.BlockSpec((1, D, C), lambda e: (e, 0, 1)),
            pl.BlockSpec((1, D, C), lambda e: (e, 0, 2)),
            pl.BlockSpec((1, D, C), lambda e: (e, 0, 3)),
            pl.BlockSpec((1, 1, 2 * F), lambda e: (e, 0, 0)),
            pl.BlockSpec((1, C, D), lambda e: (e, 0, 0)),
            pl.BlockSpec((1, C, D), lambda e: (e, 1, 0)),
            pl.BlockSpec((1, 1, D), lambda e: (e, 0, 0)),
            pl.BlockSpec((1, 1, F), lambda e: (e, 0, 0)),
            pl.BlockSpec((T, K), lambda e: (0, 0)),
        ],
        out_specs=pl.BlockSpec((T, D), lambda e: (0, 0)),
        out_shape=jax.ShapeDtypeStruct((T, D), jnp.float32),
    )(x, expert_ids, gmm1_weight, gmm1_weight, gmm1_weight, gmm1_weight,
      gmm1_weight_scale[:, None, :], gmm2_weight, gmm2_weight,
      gmm2_weight_scale[:, None, :], smooth_scales[:, None, :], expert_scales)
